# baseline (speedup 1.0000x reference)
import jax
import jax.numpy as jnp
from jax import lax
from jax.experimental import pallas as pl
from jax.experimental.pallas import tpu as pltpu
from jax.experimental.pallas import tpu_sc as plsc

N = 50000
E = 800000
H = 2
C = 32
HC = H * C
D_IN = 2
D_E = 2

NC = 2
NS = 16
NW = NC * NS
GCH = 512
SEG_CH = 512
E_FULL = E + N
E_PAD = 851968
PER_W = E_PAD // NW
N_HALF = 25000
N_QTR = 12500
Q_ROWS = 13056
E_MLP = 819200
NPAD2 = 100096
STRIPE = NPAD2 // NS
NEG = -3.0e38


def _wid():
    return lax.axis_index("s") * NC + lax.axis_index("c")


def _sc_mesh():
    return plsc.VectorSubcoreMesh(core_axis_name="c", subcore_axis_name="s")


def _make_gather_body(per_w, chunk):
    def _gather_body(table_hbm, idx_hbm, out_hbm, idx_v, rows_v, sem):
        wid = _wid()

        def step(i, carry):
            base = wid * per_w + i * chunk
            pltpu.sync_copy(idx_hbm.at[pl.ds(base, chunk)], idx_v)
            pltpu.async_copy(table_hbm.at[idx_v], rows_v, sem).wait()
            pltpu.sync_copy(rows_v, out_hbm.at[pl.ds(base, chunk)])
            return carry

        lax.fori_loop(0, per_w // chunk, step, 0)

    return _gather_body


def _sc_gather(table, idx_pad, total, chunk):
    per_w = total // NW
    return pl.kernel(
        _make_gather_body(per_w, chunk),
        out_type=jax.ShapeDtypeStruct((total, HC), jnp.float32),
        mesh=_sc_mesh(),
        scratch_types=[
            pltpu.VMEM((chunk,), jnp.int32),
            pltpu.VMEM((chunk, HC), jnp.float32),
            pltpu.SemaphoreType.DMA,
        ],
        compiler_params=pltpu.CompilerParams(use_tc_tiling_on_sc=False, needs_layout_passes=False),
    )(table, idx_pad)


def _segmax_body(idx_hbm, alpha_hbm, out_hbm, m_loc, idx_v, a_v):
    wid = _wid()

    neg = jnp.full((16,), NEG, jnp.float32)

    def init(i, carry):
        m_loc[pl.ds(i * 16, 16)] = neg
        return carry

    lax.fori_loop(0, NPAD2 // 16, init, 0)

    zeros16 = jnp.zeros((16,), jnp.int32)
    ones16 = jnp.ones((16,), jnp.int32)
    iota16 = lax.iota(jnp.int32, 16)

    def step(i, carry):
        base = wid * PER_W + i * SEG_CH
        pltpu.sync_copy(idx_hbm.at[pl.ds(base, SEG_CH)], idx_v)
        pltpu.sync_copy(alpha_hbm.at[pl.ds(base, SEG_CH)], a_v)
        for v in range(SEG_CH // 16):
            d = idx_v[pl.ds(v * 16, 16)]
            pos = iota16 + (v * 16)
            for head in range(2):
                f = d * 2 + head
                val = plsc.load_gather(a_v, [pos, zeros16 if head == 0 else ones16])
                cur = plsc.load_gather(m_loc, [f])
                need = val > cur

                def cond(st):
                    return jnp.any(st[0])

                def body(st):
                    nd, _ = st
                    plsc.store_scatter(m_loc, [f], val, mask=nd)
                    cur2 = plsc.load_gather(m_loc, [f])
                    return (val > cur2, 0)

                lax.while_loop(cond, body, (need, 0))
        return carry

    lax.fori_loop(0, PER_W // SEG_CH, step, 0)

    pltpu.sync_copy(m_loc, out_hbm.at[wid])


def _sc_segmax(idx_pad, alpha):
    return pl.kernel(
        _segmax_body,
        out_type=jax.ShapeDtypeStruct((NW, NPAD2), jnp.float32),
        mesh=_sc_mesh(),
        scratch_types=[
            pltpu.VMEM((NPAD2,), jnp.float32),
            pltpu.VMEM((SEG_CH,), jnp.int32),
            pltpu.VMEM((SEG_CH, 2), jnp.float32),
        ],
        compiler_params=pltpu.CompilerParams(use_tc_tiling_on_sc=False, needs_layout_passes=False),
    )(idx_pad, alpha)


MAXB = 5888


def _maxred_body(parts_ref, out_ref):
    out_ref[...] = jnp.max(parts_ref[...], axis=0, keepdims=True)


def _maxred_pass(parts):
    return pl.pallas_call(
        _maxred_body,
        grid=(NPAD2 // MAXB,),
        in_specs=[pl.BlockSpec((NW, MAXB), lambda i: (0, i))],
        out_specs=pl.BlockSpec((1, MAXB), lambda i: (0, i)),
        out_shape=jax.ShapeDtypeStruct((1, NPAD2), jnp.float32),
    )(parts)


def _epass_body(idx_hbm, alpha_hbm, m_hbm, out_hbm, m_loc, idx_v, a_v, e_v):
    wid = _wid()
    pltpu.sync_copy(m_hbm.at[0], m_loc)

    zeros16 = jnp.zeros((16,), jnp.int32)
    ones16 = jnp.ones((16,), jnp.int32)
    iota16 = lax.iota(jnp.int32, 16)

    def step(i, carry):
        base = wid * PER_W + i * SEG_CH
        pltpu.sync_copy(idx_hbm.at[pl.ds(base, SEG_CH)], idx_v)
        pltpu.sync_copy(alpha_hbm.at[pl.ds(base, SEG_CH)], a_v)
        for v in range(SEG_CH // 16):
            d = idx_v[pl.ds(v * 16, 16)]
            pos = iota16 + (v * 16)
            for head in range(2):
                col = zeros16 if head == 0 else ones16
                mval = plsc.load_gather(m_loc, [d * 2 + head])
                aval = plsc.load_gather(a_v, [pos, col])
                plsc.store_scatter(e_v, [pos, col], jnp.exp(aval - mval))
        pltpu.sync_copy(e_v, out_hbm.at[pl.ds(base, SEG_CH)])
        return carry

    lax.fori_loop(0, PER_W // SEG_CH, step, 0)


def _sc_epass(idx_pad, alpha, m):
    return pl.kernel(
        _epass_body,
        out_type=jax.ShapeDtypeStruct((E_PAD, 2), jnp.float32),
        mesh=_sc_mesh(),
        scratch_types=[
            pltpu.VMEM((NPAD2,), jnp.float32),
            pltpu.VMEM((SEG_CH,), jnp.int32),
            pltpu.VMEM((SEG_CH, 2), jnp.float32),
            pltpu.VMEM((SEG_CH, 2), jnp.float32),
        ],
        compiler_params=pltpu.CompilerParams(use_tc_tiling_on_sc=False, needs_layout_passes=False),
    )(idx_pad, alpha, m)


def _scatter_body(idx_hbm, w_hbm, e_hbm, z64_hbm, z2_hbm, num_hbm, s_hbm,
                  idx_v, w_v, e_v, accum, acc2):
    cid = lax.axis_index("c")
    sid = lax.axis_index("s")
    iota16 = lax.iota(jnp.int32, 16)
    n_chunks = E_PAD // NS // GCH

    for q in range(2):
        base_node = cid * N_HALF + q * N_QTR

        zb = sid * (Q_ROWS // NS)
        pltpu.sync_copy(z64_hbm, accum.at[pl.ds(zb, GCH)])
        pltpu.sync_copy(z2_hbm, acc2.at[pl.ds(zb, GCH)])
        pltpu.sync_copy(z64_hbm.at[pl.ds(0, 304)], accum.at[pl.ds(zb + GCH, 304)])
        pltpu.sync_copy(z2_hbm.at[pl.ds(0, 304)], acc2.at[pl.ds(zb + GCH, 304)])
        plsc.subcore_barrier()

        def step(i, carry):
            base = (sid * (E_PAD // NS)) + i * GCH
            pltpu.sync_copy(idx_hbm.at[pl.ds(base, GCH)], idx_v)
            pltpu.sync_copy(w_hbm.at[pl.ds(base, GCH)], w_v)
            pltpu.sync_copy(e_hbm.at[pl.ds(base, GCH)], e_v)
            for v in range(GCH // 16):
                sl = pl.ds(v * 16, 16)
                d = idx_v[sl]
                local = d - base_node
                inb = (local >= 0) & (local < N_QTR)
                trash = (N_QTR + v * 16) + iota16
                idx_v[sl] = jnp.where(inb, local, trash)
            pltpu.sync_copy(w_v, accum.at[idx_v], add=True)
            pltpu.sync_copy(e_v, acc2.at[idx_v], add=True)
            return carry

        lax.fori_loop(0, n_chunks, step, 0)
        plsc.subcore_barrier()

        for k in range(2):
            ci = sid + k * NS

            @pl.when(ci < 25)
            def _():
                pltpu.sync_copy(accum.at[pl.ds(ci * 500, 500)],
                                num_hbm.at[pl.ds(base_node + ci * 500, 500)])
                pltpu.sync_copy(acc2.at[pl.ds(ci * 500, 500)],
                                s_hbm.at[pl.ds(base_node + ci * 500, 500)])
        if q == 0:
            plsc.subcore_barrier()


def _sc_scatter(idx_pad, w, e2):
    z64 = jnp.zeros((GCH, HC), jnp.float32)
    z2 = jnp.zeros((GCH, 2), jnp.float32)
    return pl.kernel(
        _scatter_body,
        out_type=(jax.ShapeDtypeStruct((N, HC), jnp.float32),
                  jax.ShapeDtypeStruct((N, 2), jnp.float32)),
        mesh=_sc_mesh(),
        scratch_types=[
            pltpu.VMEM((GCH,), jnp.int32),
            pltpu.VMEM((GCH, HC), jnp.float32),
            pltpu.VMEM((GCH, 2), jnp.float32),
            pltpu.VMEM_SHARED((Q_ROWS, HC), jnp.float32),
            pltpu.VMEM_SHARED((Q_ROWS, 2), jnp.float32),
        ],
        compiler_params=pltpu.CompilerParams(use_tc_tiling_on_sc=False, needs_layout_passes=False),
    )(idx_pad, w, e2, z64, z2)


XL_R = 2000


def _xl_body(h_ref, w_ref, b_ref, out_ref):
    out_ref[...] = jnp.dot(h_ref[...], w_ref[...],
                           preferred_element_type=jnp.float32) + b_ref[...]


def _xl_pass(h, W, b):
    d_in = h.shape[1]
    return pl.pallas_call(
        _xl_body,
        grid=(N // XL_R,),
        in_specs=[
            pl.BlockSpec((XL_R, d_in), lambda i: (i, 0)),
            pl.BlockSpec((d_in, HC), lambda i: (0, 0)),
            pl.BlockSpec((1, HC), lambda i: (0, 0)),
        ],
        out_specs=pl.BlockSpec((XL_R, HC), lambda i: (i, 0)),
        out_shape=jax.ShapeDtypeStruct((N, HC), jnp.float32),
    )(h, W, b.reshape(1, HC))


AL_R = 8192


def _alpha_body(xs_ref, xd_ref, ea_ref, we_ref, att_ref, out_ref):
    i = pl.program_id(0)
    q = xs_ref[...] + xd_ref[...] + jnp.dot(ea_ref[...], we_ref[...],
                                            preferred_element_type=jnp.float32)
    g = jnp.where(q > 0, q, 0.2 * q)
    ga = g * att_ref[...]
    a0 = jnp.sum(ga[:, :C], axis=1, keepdims=True)
    a1 = jnp.sum(ga[:, C:], axis=1, keepdims=True)
    alpha = jnp.concatenate([a0, a1], axis=1)
    row = i * AL_R + lax.broadcasted_iota(jnp.int32, (AL_R, 2), 0)
    out_ref[...] = jnp.where(row < E_FULL, alpha, NEG)


def _alpha_pass(xs, xd, ea_pad, We, att):
    return pl.pallas_call(
        _alpha_body,
        grid=(E_PAD // AL_R,),
        in_specs=[
            pl.BlockSpec((AL_R, HC), lambda i: (i, 0)),
            pl.BlockSpec((AL_R, HC), lambda i: (i, 0)),
            pl.BlockSpec((AL_R, D_E), lambda i: (i, 0)),
            pl.BlockSpec((D_E, HC), lambda i: (0, 0)),
            pl.BlockSpec((1, HC), lambda i: (0, 0)),
        ],
        out_specs=pl.BlockSpec((AL_R, 2), lambda i: (i, 0)),
        out_shape=jax.ShapeDtypeStruct((E_PAD, 2), jnp.float32),
    )(xs, xd, ea_pad, We, att.reshape(1, HC))


def _w_body(xs_ref, e_ref, out_ref):
    xs = xs_ref[...]
    e = e_ref[...]
    e0 = jnp.broadcast_to(e[:, 0:1], (AL_R, C))
    e1 = jnp.broadcast_to(e[:, 1:2], (AL_R, C))
    out_ref[...] = xs * jnp.concatenate([e0, e1], axis=1)


def _w_pass(xs, e2):
    return pl.pallas_call(
        _w_body,
        grid=(E_PAD // AL_R,),
        in_specs=[
            pl.BlockSpec((AL_R, HC), lambda i: (i, 0)),
            pl.BlockSpec((AL_R, 2), lambda i: (i, 0)),
        ],
        out_specs=pl.BlockSpec((AL_R, HC), lambda i: (i, 0)),
        out_shape=jax.ShapeDtypeStruct((E_PAD, HC), jnp.float32),
    )(xs, e2)


def _combine_body(num_ref, s_ref, b_ref, out_ref):
    num = num_ref[...]
    s = s_ref[...]
    s0 = jnp.broadcast_to(s[:, 0:1], (XL_R, C))
    s1 = jnp.broadcast_to(s[:, 1:2], (XL_R, C))
    o = num / (jnp.concatenate([s0, s1], axis=1) + 1e-16) + b_ref[...]
    out_ref[...] = jnp.where(o > 0, o, jnp.exp(o) - 1.0)


def _combine_pass(num, s, bias):
    return pl.pallas_call(
        _combine_body,
        grid=(N // XL_R,),
        in_specs=[
            pl.BlockSpec((XL_R, HC), lambda i: (i, 0)),
            pl.BlockSpec((XL_R, 2), lambda i: (i, 0)),
            pl.BlockSpec((1, HC), lambda i: (0, 0)),
        ],
        out_specs=pl.BlockSpec((XL_R, HC), lambda i: (i, 0)),
        out_shape=jax.ShapeDtypeStruct((N, HC), jnp.float32),
    )(num, s, bias.reshape(1, HC))


def _mean_body(ea_ref, out_ref):
    @pl.when(pl.program_id(0) == 0)
    def _():
        out_ref[...] = jnp.zeros_like(out_ref)

    out_ref[...] += jnp.sum(ea_ref[...], axis=0, keepdims=True) * (1.0 / E)


def _mean_pass(ea):
    return pl.pallas_call(
        _mean_body,
        grid=(100,),
        in_specs=[pl.BlockSpec((E // 100, D_E), lambda i: (i, 0))],
        out_specs=pl.BlockSpec((1, D_E), lambda i: (0, 0)),
        out_shape=jax.ShapeDtypeStruct((1, D_E), jnp.float32),
    )(ea)


MLP_R = 8192


def _mlp_body(hs_ref, hd_ref, ea_ref, wa_ref, wb_ref, wd_ref, bias_ref, w2_ref, b2_ref, out_ref):
    acc = jnp.dot(hs_ref[...], wa_ref[...], preferred_element_type=jnp.float32)
    acc += jnp.dot(hd_ref[...], wb_ref[...], preferred_element_type=jnp.float32)
    acc += jnp.dot(ea_ref[...], wd_ref[...], preferred_element_type=jnp.float32)
    hid = jnp.maximum(acc + bias_ref[...], 0.0)
    res = jnp.dot(hid, w2_ref[...], preferred_element_type=jnp.float32)[:, 0] + b2_ref[0]
    out_ref[...] = res.reshape(1, 1, MLP_R)


def _edge_mlp(h_src, h_dst, ea, goal, Wm1, bm1, Wm2, bm2):
    wa = Wm1[0:HC]
    wb = Wm1[HC:2 * HC]
    wc = Wm1[2 * HC:3 * HC]
    wd = Wm1[3 * HC:]
    bias_eff = (bm1 + goal @ wc).reshape(1, 32)
    return pl.pallas_call(
        _mlp_body,
        grid=(E_MLP // MLP_R,),
        in_specs=[
            pl.BlockSpec((MLP_R, HC), lambda i: (i, 0)),
            pl.BlockSpec((MLP_R, HC), lambda i: (i, 0)),
            pl.BlockSpec((MLP_R, D_E), lambda i: (i, 0)),
            pl.BlockSpec((HC, 32), lambda i: (0, 0)),
            pl.BlockSpec((HC, 32), lambda i: (0, 0)),
            pl.BlockSpec((D_E, 32), lambda i: (0, 0)),
            pl.BlockSpec((1, 32), lambda i: (0, 0)),
            pl.BlockSpec((32, 1), lambda i: (0, 0)),
            pl.BlockSpec((1,), lambda i: (0,)),
        ],
        out_specs=pl.BlockSpec((1, 1, MLP_R), lambda i: (i, 0, 0)),
        out_shape=jax.ShapeDtypeStruct((E_MLP // MLP_R, 1, MLP_R), jnp.float32),
    )(h_src, h_dst, ea, wa, wb, wd, bias_eff, Wm2, bm2).reshape(E_MLP)


def _gatv2_layer(xl2d, src_pad, dst_pad, ea_pad, We, att, bias):
    xs = _sc_gather(xl2d, src_pad, E_PAD, GCH)
    xd = _sc_gather(xl2d, dst_pad, E_PAD, GCH)
    alpha = _alpha_pass(xs, xd, ea_pad, We, att)
    mparts = _sc_segmax(dst_pad, alpha)
    m = _maxred_pass(mparts)
    e2 = _sc_epass(dst_pad, alpha, m)
    w = _w_pass(xs, e2)
    num, s = _sc_scatter(dst_pad, w, e2)
    return _combine_pass(num, s, bias)


def kernel(x, edge_index, edge_attr, dest_index, W1_l, b1_l, We1, att1, bias1,
           W2_l, b2_l, We2, att2, bias2, Wm1, bm1, Wm2, bm2):
    src, dst = edge_index[0], edge_index[1]
    loop = jnp.arange(N, dtype=src.dtype)
    pad = (jnp.arange(E_PAD - E_FULL, dtype=src.dtype) * 61) % N
    src_pad = jnp.concatenate([src, loop, pad], axis=0)
    dst_pad = jnp.concatenate([dst, loop, pad], axis=0)
    ea_mean = _mean_pass(edge_attr)
    ea_pad = jnp.concatenate([
        edge_attr,
        jnp.broadcast_to(ea_mean, (N, D_E)),
        jnp.zeros((E_PAD - E_FULL, D_E), jnp.float32),
    ], axis=0)

    xl1 = _xl_pass(x, W1_l, b1_l)
    h = _gatv2_layer(xl1, src_pad, dst_pad, ea_pad, We1, att1.reshape(HC), bias1)
    xl2 = _xl_pass(h, W2_l, b2_l)
    h = _gatv2_layer(xl2, src_pad, dst_pad, ea_pad, We2, att2.reshape(HC), bias2)

    goal = h[dest_index]
    mpad = (jnp.arange(E_MLP - E, dtype=src.dtype) * 61) % N
    src_m = jnp.concatenate([src, mpad], axis=0)
    dst_m = jnp.concatenate([dst, mpad], axis=0)
    hs = _sc_gather(h, src_m, E_MLP, GCH)
    hd = _sc_gather(h, dst_m, E_MLP, GCH)
    ea_m = jnp.concatenate([edge_attr, jnp.zeros((E_MLP - E, D_E), jnp.float32)], axis=0)
    return _edge_mlp(hs, hd, ea_m, goal, Wm1, bm1, Wm2, bm2)[:E]


# speedup vs baseline: 1.0371x; 1.0371x over previous
import jax
import jax.numpy as jnp
from jax import lax
from jax.experimental import pallas as pl
from jax.experimental.pallas import tpu as pltpu
from jax.experimental.pallas import tpu_sc as plsc

N = 50000
E = 800000
H = 2
C = 32
HC = H * C
D_IN = 2
D_E = 2

NC = 2
NS = 16
NW = NC * NS
GCH = 512
SEG_CH = 512
E_FULL = E + N
E_PAD = 851968
PER_W = E_PAD // NW
N_HALF = 25000
N_QTR = 12500
Q_ROWS = 13056
E_MLP = 819200
NPAD2 = 100096
STRIPE = NPAD2 // NS
NEG = -3.0e38


def _wid():
    return lax.axis_index("s") * NC + lax.axis_index("c")


def _sc_mesh():
    return plsc.VectorSubcoreMesh(core_axis_name="c", subcore_axis_name="s")


def _make_gather_body(per_w, chunk):
    def _gather_body(table_hbm, idx_hbm, out_hbm, idx_v, rows_v, sem):
        wid = _wid()

        def step(i, carry):
            base = wid * per_w + i * chunk
            pltpu.sync_copy(idx_hbm.at[pl.ds(base, chunk)], idx_v)
            pltpu.async_copy(table_hbm.at[idx_v], rows_v, sem).wait()
            pltpu.sync_copy(rows_v, out_hbm.at[pl.ds(base, chunk)])
            return carry

        lax.fori_loop(0, per_w // chunk, step, 0)

    return _gather_body


def _sc_gather(table, idx_pad, total, chunk):
    per_w = total // NW
    return pl.kernel(
        _make_gather_body(per_w, chunk),
        out_type=jax.ShapeDtypeStruct((total, HC), jnp.float32),
        mesh=_sc_mesh(),
        scratch_types=[
            pltpu.VMEM((chunk,), jnp.int32),
            pltpu.VMEM((chunk, HC), jnp.float32),
            pltpu.SemaphoreType.DMA,
        ],
        compiler_params=pltpu.CompilerParams(use_tc_tiling_on_sc=False, needs_layout_passes=False),
    )(table, idx_pad)


def _segmax_body(idx_hbm, alpha_hbm, out_hbm, m_loc, idx_v, a_v):
    wid = _wid()

    neg = jnp.full((16,), NEG, jnp.float32)

    def init(i, carry):
        m_loc[pl.ds(i * 16, 16)] = neg
        return carry

    lax.fori_loop(0, NPAD2 // 16, init, 0)

    zeros16 = jnp.zeros((16,), jnp.int32)
    ones16 = jnp.ones((16,), jnp.int32)
    iota16 = lax.iota(jnp.int32, 16)

    def step(i, carry):
        base = wid * PER_W + i * SEG_CH
        pltpu.sync_copy(idx_hbm.at[pl.ds(base, SEG_CH)], idx_v)
        pltpu.sync_copy(alpha_hbm.at[pl.ds(base, SEG_CH)], a_v)
        for v in range(SEG_CH // 16):
            d = idx_v[pl.ds(v * 16, 16)]
            pos = iota16 + (v * 16)
            for head in range(2):
                f = d * 2 + head
                val = plsc.load_gather(a_v, [pos, zeros16 if head == 0 else ones16])
                cur = plsc.load_gather(m_loc, [f])
                need = val > cur

                def cond(st):
                    return jnp.any(st[0])

                def body(st):
                    nd, _ = st
                    plsc.store_scatter(m_loc, [f], val, mask=nd)
                    cur2 = plsc.load_gather(m_loc, [f])
                    return (val > cur2, 0)

                lax.while_loop(cond, body, (need, 0))
        return carry

    lax.fori_loop(0, PER_W // SEG_CH, step, 0)

    pltpu.sync_copy(m_loc, out_hbm.at[wid])


def _sc_segmax(idx_pad, alpha):
    return pl.kernel(
        _segmax_body,
        out_type=jax.ShapeDtypeStruct((NW, NPAD2), jnp.float32),
        mesh=_sc_mesh(),
        scratch_types=[
            pltpu.VMEM((NPAD2,), jnp.float32),
            pltpu.VMEM((SEG_CH,), jnp.int32),
            pltpu.VMEM((SEG_CH, 2), jnp.float32),
        ],
        compiler_params=pltpu.CompilerParams(use_tc_tiling_on_sc=False, needs_layout_passes=False),
    )(idx_pad, alpha)


MAXB = 5888


def _maxred_body(parts_ref, out_ref):
    out_ref[...] = jnp.max(parts_ref[...], axis=0, keepdims=True)


def _maxred_pass(parts):
    return pl.pallas_call(
        _maxred_body,
        grid=(NPAD2 // MAXB,),
        in_specs=[pl.BlockSpec((NW, MAXB), lambda i: (0, i))],
        out_specs=pl.BlockSpec((1, MAXB), lambda i: (0, i)),
        out_shape=jax.ShapeDtypeStruct((1, NPAD2), jnp.float32),
    )(parts)


def _epass_body(idx_hbm, alpha_hbm, m_hbm, out_hbm, m_loc, idx_v, a_v, e_v):
    wid = _wid()
    pltpu.sync_copy(m_hbm.at[0], m_loc)

    zeros16 = jnp.zeros((16,), jnp.int32)
    ones16 = jnp.ones((16,), jnp.int32)
    iota16 = lax.iota(jnp.int32, 16)

    def step(i, carry):
        base = wid * PER_W + i * SEG_CH
        pltpu.sync_copy(idx_hbm.at[pl.ds(base, SEG_CH)], idx_v)
        pltpu.sync_copy(alpha_hbm.at[pl.ds(base, SEG_CH)], a_v)
        for v in range(SEG_CH // 16):
            d = idx_v[pl.ds(v * 16, 16)]
            pos = iota16 + (v * 16)
            for head in range(2):
                col = zeros16 if head == 0 else ones16
                mval = plsc.load_gather(m_loc, [d * 2 + head])
                aval = plsc.load_gather(a_v, [pos, col])
                plsc.store_scatter(e_v, [pos, col], jnp.exp(aval - mval))
        pltpu.sync_copy(e_v, out_hbm.at[pl.ds(base, SEG_CH)])
        return carry

    lax.fori_loop(0, PER_W // SEG_CH, step, 0)


def _sc_epass(idx_pad, alpha, m):
    return pl.kernel(
        _epass_body,
        out_type=jax.ShapeDtypeStruct((E_PAD, 2), jnp.float32),
        mesh=_sc_mesh(),
        scratch_types=[
            pltpu.VMEM((NPAD2,), jnp.float32),
            pltpu.VMEM((SEG_CH,), jnp.int32),
            pltpu.VMEM((SEG_CH, 2), jnp.float32),
            pltpu.VMEM((SEG_CH, 2), jnp.float32),
        ],
        compiler_params=pltpu.CompilerParams(use_tc_tiling_on_sc=False, needs_layout_passes=False),
    )(idx_pad, alpha, m)


def _scatter_body(idx_hbm, w_hbm, e_hbm, z64_hbm, z2_hbm, num_hbm, s_hbm,
                  idx_v, w_v, e_v, accum, acc2, sem, sem2, sem3):
    cid = lax.axis_index("c")
    sid = lax.axis_index("s")
    iota16 = lax.iota(jnp.int32, 16)
    n_chunks = E_PAD // NS // GCH

    for q in range(2):
        base_node = cid * N_HALF + q * N_QTR

        zb = sid * (Q_ROWS // NS)
        pltpu.sync_copy(z64_hbm, accum.at[pl.ds(zb, GCH)])
        pltpu.sync_copy(z2_hbm, acc2.at[pl.ds(zb, GCH)])
        pltpu.sync_copy(z64_hbm.at[pl.ds(0, 304)], accum.at[pl.ds(zb + GCH, 304)])
        pltpu.sync_copy(z2_hbm.at[pl.ds(0, 304)], acc2.at[pl.ds(zb + GCH, 304)])
        plsc.subcore_barrier()

        def step(i, carry):
            base = (sid * (E_PAD // NS)) + i * GCH
            h1 = pltpu.async_copy(idx_hbm.at[pl.ds(base, GCH)], idx_v, sem)
            h2 = pltpu.async_copy(w_hbm.at[pl.ds(base, GCH)], w_v, sem2)
            h3 = pltpu.async_copy(e_hbm.at[pl.ds(base, GCH)], e_v, sem3)
            h1.wait()
            h2.wait()
            h3.wait()
            for v in range(GCH // 16):
                sl = pl.ds(v * 16, 16)
                d = idx_v[sl]
                local = d - base_node
                inb = (local >= 0) & (local < N_QTR)
                trash = (N_QTR + v * 16) + iota16
                idx_v[sl] = jnp.where(inb, local, trash)
            pltpu.sync_copy(w_v, accum.at[idx_v], add=True)
            pltpu.sync_copy(e_v, acc2.at[idx_v], add=True)
            return carry

        lax.fori_loop(0, n_chunks, step, 0)
        plsc.subcore_barrier()

        for k in range(2):
            ci = sid + k * NS

            @pl.when(ci < 25)
            def _():
                pltpu.sync_copy(accum.at[pl.ds(ci * 500, 500)],
                                num_hbm.at[pl.ds(base_node + ci * 500, 500)])
                pltpu.sync_copy(acc2.at[pl.ds(ci * 500, 500)],
                                s_hbm.at[pl.ds(base_node + ci * 500, 500)])
        if q == 0:
            plsc.subcore_barrier()


def _sc_scatter(idx_pad, w, e2):
    z64 = jnp.zeros((GCH, HC), jnp.float32)
    z2 = jnp.zeros((GCH, 2), jnp.float32)
    return pl.kernel(
        _scatter_body,
        out_type=(jax.ShapeDtypeStruct((N, HC), jnp.float32),
                  jax.ShapeDtypeStruct((N, 2), jnp.float32)),
        mesh=_sc_mesh(),
        scratch_types=[
            pltpu.VMEM((GCH,), jnp.int32),
            pltpu.VMEM((GCH, HC), jnp.float32),
            pltpu.VMEM((GCH, 2), jnp.float32),
            pltpu.VMEM_SHARED((Q_ROWS, HC), jnp.float32),
            pltpu.VMEM_SHARED((Q_ROWS, 2), jnp.float32),
            pltpu.SemaphoreType.DMA,
            pltpu.SemaphoreType.DMA,
            pltpu.SemaphoreType.DMA,
        ],
        compiler_params=pltpu.CompilerParams(use_tc_tiling_on_sc=False, needs_layout_passes=False),
    )(idx_pad, w, e2, z64, z2)


XL_R = 2000


def _xl_body(h_ref, w_ref, b_ref, out_ref):
    out_ref[...] = jnp.dot(h_ref[...], w_ref[...],
                           preferred_element_type=jnp.float32) + b_ref[...]


def _xl_pass(h, W, b):
    d_in = h.shape[1]
    return pl.pallas_call(
        _xl_body,
        grid=(N // XL_R,),
        in_specs=[
            pl.BlockSpec((XL_R, d_in), lambda i: (i, 0)),
            pl.BlockSpec((d_in, HC), lambda i: (0, 0)),
            pl.BlockSpec((1, HC), lambda i: (0, 0)),
        ],
        out_specs=pl.BlockSpec((XL_R, HC), lambda i: (i, 0)),
        out_shape=jax.ShapeDtypeStruct((N, HC), jnp.float32),
    )(h, W, b.reshape(1, HC))


AL_R = 8192


def _alpha_body(xs_ref, xd_ref, ea_ref, we_ref, att_ref, out_ref):
    i = pl.program_id(0)
    q = xs_ref[...] + xd_ref[...] + jnp.dot(ea_ref[...], we_ref[...],
                                            preferred_element_type=jnp.float32)
    g = jnp.where(q > 0, q, 0.2 * q)
    ga = g * att_ref[...]
    a0 = jnp.sum(ga[:, :C], axis=1, keepdims=True)
    a1 = jnp.sum(ga[:, C:], axis=1, keepdims=True)
    alpha = jnp.concatenate([a0, a1], axis=1)
    row = i * AL_R + lax.broadcasted_iota(jnp.int32, (AL_R, 2), 0)
    out_ref[...] = jnp.where(row < E_FULL, alpha, NEG)


def _alpha_pass(xs, xd, ea_pad, We, att):
    return pl.pallas_call(
        _alpha_body,
        grid=(E_PAD // AL_R,),
        in_specs=[
            pl.BlockSpec((AL_R, HC), lambda i: (i, 0)),
            pl.BlockSpec((AL_R, HC), lambda i: (i, 0)),
            pl.BlockSpec((AL_R, D_E), lambda i: (i, 0)),
            pl.BlockSpec((D_E, HC), lambda i: (0, 0)),
            pl.BlockSpec((1, HC), lambda i: (0, 0)),
        ],
        out_specs=pl.BlockSpec((AL_R, 2), lambda i: (i, 0)),
        out_shape=jax.ShapeDtypeStruct((E_PAD, 2), jnp.float32),
    )(xs, xd, ea_pad, We, att.reshape(1, HC))


def _w_body(xs_ref, e_ref, out_ref):
    xs = xs_ref[...]
    e = e_ref[...]
    e0 = jnp.broadcast_to(e[:, 0:1], (AL_R, C))
    e1 = jnp.broadcast_to(e[:, 1:2], (AL_R, C))
    out_ref[...] = xs * jnp.concatenate([e0, e1], axis=1)


def _w_pass(xs, e2):
    return pl.pallas_call(
        _w_body,
        grid=(E_PAD // AL_R,),
        in_specs=[
            pl.BlockSpec((AL_R, HC), lambda i: (i, 0)),
            pl.BlockSpec((AL_R, 2), lambda i: (i, 0)),
        ],
        out_specs=pl.BlockSpec((AL_R, HC), lambda i: (i, 0)),
        out_shape=jax.ShapeDtypeStruct((E_PAD, HC), jnp.float32),
    )(xs, e2)


def _combine_body(num_ref, s_ref, b_ref, out_ref):
    num = num_ref[...]
    s = s_ref[...]
    s0 = jnp.broadcast_to(s[:, 0:1], (XL_R, C))
    s1 = jnp.broadcast_to(s[:, 1:2], (XL_R, C))
    o = num / (jnp.concatenate([s0, s1], axis=1) + 1e-16) + b_ref[...]
    out_ref[...] = jnp.where(o > 0, o, jnp.exp(o) - 1.0)


def _combine_pass(num, s, bias):
    return pl.pallas_call(
        _combine_body,
        grid=(N // XL_R,),
        in_specs=[
            pl.BlockSpec((XL_R, HC), lambda i: (i, 0)),
            pl.BlockSpec((XL_R, 2), lambda i: (i, 0)),
            pl.BlockSpec((1, HC), lambda i: (0, 0)),
        ],
        out_specs=pl.BlockSpec((XL_R, HC), lambda i: (i, 0)),
        out_shape=jax.ShapeDtypeStruct((N, HC), jnp.float32),
    )(num, s, bias.reshape(1, HC))


def _mean_body(ea_ref, out_ref):
    @pl.when(pl.program_id(0) == 0)
    def _():
        out_ref[...] = jnp.zeros_like(out_ref)

    out_ref[...] += jnp.sum(ea_ref[...], axis=0, keepdims=True) * (1.0 / E)


def _mean_pass(ea):
    return pl.pallas_call(
        _mean_body,
        grid=(100,),
        in_specs=[pl.BlockSpec((E // 100, D_E), lambda i: (i, 0))],
        out_specs=pl.BlockSpec((1, D_E), lambda i: (0, 0)),
        out_shape=jax.ShapeDtypeStruct((1, D_E), jnp.float32),
    )(ea)


MLP_R = 8192


def _mlp_body(hs_ref, hd_ref, ea_ref, wa_ref, wb_ref, wd_ref, bias_ref, w2_ref, b2_ref, out_ref):
    acc = jnp.dot(hs_ref[...], wa_ref[...], preferred_element_type=jnp.float32)
    acc += jnp.dot(hd_ref[...], wb_ref[...], preferred_element_type=jnp.float32)
    acc += jnp.dot(ea_ref[...], wd_ref[...], preferred_element_type=jnp.float32)
    hid = jnp.maximum(acc + bias_ref[...], 0.0)
    res = jnp.dot(hid, w2_ref[...], preferred_element_type=jnp.float32)[:, 0] + b2_ref[0]
    out_ref[...] = res.reshape(1, 1, MLP_R)


def _edge_mlp(h_src, h_dst, ea, goal, Wm1, bm1, Wm2, bm2):
    wa = Wm1[0:HC]
    wb = Wm1[HC:2 * HC]
    wc = Wm1[2 * HC:3 * HC]
    wd = Wm1[3 * HC:]
    bias_eff = (bm1 + goal @ wc).reshape(1, 32)
    return pl.pallas_call(
        _mlp_body,
        grid=(E_MLP // MLP_R,),
        in_specs=[
            pl.BlockSpec((MLP_R, HC), lambda i: (i, 0)),
            pl.BlockSpec((MLP_R, HC), lambda i: (i, 0)),
            pl.BlockSpec((MLP_R, D_E), lambda i: (i, 0)),
            pl.BlockSpec((HC, 32), lambda i: (0, 0)),
            pl.BlockSpec((HC, 32), lambda i: (0, 0)),
            pl.BlockSpec((D_E, 32), lambda i: (0, 0)),
            pl.BlockSpec((1, 32), lambda i: (0, 0)),
            pl.BlockSpec((32, 1), lambda i: (0, 0)),
            pl.BlockSpec((1,), lambda i: (0,)),
        ],
        out_specs=pl.BlockSpec((1, 1, MLP_R), lambda i: (i, 0, 0)),
        out_shape=jax.ShapeDtypeStruct((E_MLP // MLP_R, 1, MLP_R), jnp.float32),
    )(h_src, h_dst, ea, wa, wb, wd, bias_eff, Wm2, bm2).reshape(E_MLP)


def _gatv2_layer(xl2d, src_pad, dst_pad, ea_pad, We, att, bias):
    xs = _sc_gather(xl2d, src_pad, E_PAD, GCH)
    xd = _sc_gather(xl2d, dst_pad, E_PAD, GCH)
    alpha = _alpha_pass(xs, xd, ea_pad, We, att)
    mparts = _sc_segmax(dst_pad, alpha)
    m = _maxred_pass(mparts)
    e2 = _sc_epass(dst_pad, alpha, m)
    w = _w_pass(xs, e2)
    num, s = _sc_scatter(dst_pad, w, e2)
    return _combine_pass(num, s, bias)


def kernel(x, edge_index, edge_attr, dest_index, W1_l, b1_l, We1, att1, bias1,
           W2_l, b2_l, We2, att2, bias2, Wm1, bm1, Wm2, bm2):
    src, dst = edge_index[0], edge_index[1]
    loop = jnp.arange(N, dtype=src.dtype)
    pad = (jnp.arange(E_PAD - E_FULL, dtype=src.dtype) * 61) % N
    src_pad = jnp.concatenate([src, loop, pad], axis=0)
    dst_pad = jnp.concatenate([dst, loop, pad], axis=0)
    ea_mean = _mean_pass(edge_attr)
    ea_pad = jnp.concatenate([
        edge_attr,
        jnp.broadcast_to(ea_mean, (N, D_E)),
        jnp.zeros((E_PAD - E_FULL, D_E), jnp.float32),
    ], axis=0)

    xl1 = _xl_pass(x, W1_l, b1_l)
    h = _gatv2_layer(xl1, src_pad, dst_pad, ea_pad, We1, att1.reshape(HC), bias1)
    xl2 = _xl_pass(h, W2_l, b2_l)
    h = _gatv2_layer(xl2, src_pad, dst_pad, ea_pad, We2, att2.reshape(HC), bias2)

    goal = h[dest_index]
    mpad = (jnp.arange(E_MLP - E, dtype=src.dtype) * 61) % N
    src_m = jnp.concatenate([src, mpad], axis=0)
    dst_m = jnp.concatenate([dst, mpad], axis=0)
    hs = _sc_gather(h, src_m, E_MLP, GCH)
    hd = _sc_gather(h, dst_m, E_MLP, GCH)
    ea_m = jnp.concatenate([edge_attr, jnp.zeros((E_MLP - E, D_E), jnp.float32)], axis=0)
    return _edge_mlp(hs, hd, ea_m, goal, Wm1, bm1, Wm2, bm2)[:E]


# speedup vs baseline: 1.0474x; 1.0099x over previous
import jax
import jax.numpy as jnp
from jax import lax
from jax.experimental import pallas as pl
from jax.experimental.pallas import tpu as pltpu
from jax.experimental.pallas import tpu_sc as plsc

N = 50000
E = 800000
H = 2
C = 32
HC = H * C
D_IN = 2
D_E = 2

NC = 2
NS = 16
NW = NC * NS
GCH = 512
SEG_CH = 512
E_FULL = E + N
E_PAD = 851968
PER_W = E_PAD // NW
N_HALF = 25000
N_QTR = 12500
Q_ROWS = 13056
E_MLP = 819200
NPAD2 = 100096
STRIPE = NPAD2 // NS
NEG = -3.0e38


def _wid():
    return lax.axis_index("s") * NC + lax.axis_index("c")


def _sc_mesh():
    return plsc.VectorSubcoreMesh(core_axis_name="c", subcore_axis_name="s")


def _make_gather_body(per_w, chunk):
    def _gather_body(table_hbm, idx_hbm, out_hbm, idx_v, rows_v, sem):
        wid = _wid()

        def step(i, carry):
            base = wid * per_w + i * chunk
            pltpu.sync_copy(idx_hbm.at[pl.ds(base, chunk)], idx_v)
            pltpu.async_copy(table_hbm.at[idx_v], rows_v, sem).wait()
            pltpu.sync_copy(rows_v, out_hbm.at[pl.ds(base, chunk)])
            return carry

        lax.fori_loop(0, per_w // chunk, step, 0)

    return _gather_body


def _sc_gather(table, idx_pad, total, chunk):
    per_w = total // NW
    return pl.kernel(
        _make_gather_body(per_w, chunk),
        out_type=jax.ShapeDtypeStruct((total, HC), jnp.float32),
        mesh=_sc_mesh(),
        scratch_types=[
            pltpu.VMEM((chunk,), jnp.int32),
            pltpu.VMEM((chunk, HC), jnp.float32),
            pltpu.SemaphoreType.DMA,
        ],
        compiler_params=pltpu.CompilerParams(use_tc_tiling_on_sc=False, needs_layout_passes=False),
    )(table, idx_pad)


def _segmax_body(idx_hbm, alpha_hbm, out_hbm, m_loc, idx_v, a_v, sem, sem2):
    wid = _wid()

    neg = jnp.full((16,), NEG, jnp.float32)

    def init(i, carry):
        m_loc[pl.ds(i * 16, 16)] = neg
        return carry

    lax.fori_loop(0, NPAD2 // 16, init, 0)

    zeros16 = jnp.zeros((16,), jnp.int32)
    ones16 = jnp.ones((16,), jnp.int32)
    iota16 = lax.iota(jnp.int32, 16)

    def step(i, carry):
        base = wid * PER_W + i * SEG_CH
        h1 = pltpu.async_copy(idx_hbm.at[pl.ds(base, SEG_CH)], idx_v, sem)
        h2 = pltpu.async_copy(alpha_hbm.at[pl.ds(base, SEG_CH)], a_v, sem2)
        h1.wait()
        h2.wait()
        for v in range(SEG_CH // 16):
            d = idx_v[pl.ds(v * 16, 16)]
            pos = iota16 + (v * 16)
            for head in range(2):
                f = d * 2 + head
                val = plsc.load_gather(a_v, [pos, zeros16 if head == 0 else ones16])
                cur = plsc.load_gather(m_loc, [f])
                need = val > cur

                def cond(st):
                    return jnp.any(st[0])

                def body(st):
                    nd, _ = st
                    plsc.store_scatter(m_loc, [f], val, mask=nd)
                    cur2 = plsc.load_gather(m_loc, [f])
                    return (val > cur2, 0)

                lax.while_loop(cond, body, (need, 0))
        return carry

    lax.fori_loop(0, PER_W // SEG_CH, step, 0)

    pltpu.sync_copy(m_loc, out_hbm.at[wid])


def _sc_segmax(idx_pad, alpha):
    return pl.kernel(
        _segmax_body,
        out_type=jax.ShapeDtypeStruct((NW, NPAD2), jnp.float32),
        mesh=_sc_mesh(),
        scratch_types=[
            pltpu.VMEM((NPAD2,), jnp.float32),
            pltpu.VMEM((SEG_CH,), jnp.int32),
            pltpu.VMEM((SEG_CH, 2), jnp.float32),
            pltpu.SemaphoreType.DMA,
            pltpu.SemaphoreType.DMA,
        ],
        compiler_params=pltpu.CompilerParams(use_tc_tiling_on_sc=False, needs_layout_passes=False),
    )(idx_pad, alpha)


MAXB = 5888


def _maxred_body(parts_ref, out_ref):
    out_ref[...] = jnp.max(parts_ref[...], axis=0, keepdims=True)


def _maxred_pass(parts):
    return pl.pallas_call(
        _maxred_body,
        grid=(NPAD2 // MAXB,),
        in_specs=[pl.BlockSpec((NW, MAXB), lambda i: (0, i))],
        out_specs=pl.BlockSpec((1, MAXB), lambda i: (0, i)),
        out_shape=jax.ShapeDtypeStruct((1, NPAD2), jnp.float32),
    )(parts)


def _epass_body(idx_hbm, alpha_hbm, m_hbm, out_hbm, m_loc, idx_v, a_v, e_v, sem, sem2):
    wid = _wid()
    pltpu.sync_copy(m_hbm.at[0], m_loc)

    zeros16 = jnp.zeros((16,), jnp.int32)
    ones16 = jnp.ones((16,), jnp.int32)
    iota16 = lax.iota(jnp.int32, 16)

    def step(i, carry):
        base = wid * PER_W + i * SEG_CH
        h1 = pltpu.async_copy(idx_hbm.at[pl.ds(base, SEG_CH)], idx_v, sem)
        h2 = pltpu.async_copy(alpha_hbm.at[pl.ds(base, SEG_CH)], a_v, sem2)
        h1.wait()
        h2.wait()
        for v in range(SEG_CH // 16):
            d = idx_v[pl.ds(v * 16, 16)]
            pos = iota16 + (v * 16)
            for head in range(2):
                col = zeros16 if head == 0 else ones16
                mval = plsc.load_gather(m_loc, [d * 2 + head])
                aval = plsc.load_gather(a_v, [pos, col])
                plsc.store_scatter(e_v, [pos, col], jnp.exp(aval - mval))
        pltpu.sync_copy(e_v, out_hbm.at[pl.ds(base, SEG_CH)])
        return carry

    lax.fori_loop(0, PER_W // SEG_CH, step, 0)


def _sc_epass(idx_pad, alpha, m):
    return pl.kernel(
        _epass_body,
        out_type=jax.ShapeDtypeStruct((E_PAD, 2), jnp.float32),
        mesh=_sc_mesh(),
        scratch_types=[
            pltpu.VMEM((NPAD2,), jnp.float32),
            pltpu.VMEM((SEG_CH,), jnp.int32),
            pltpu.VMEM((SEG_CH, 2), jnp.float32),
            pltpu.VMEM((SEG_CH, 2), jnp.float32),
            pltpu.SemaphoreType.DMA,
            pltpu.SemaphoreType.DMA,
        ],
        compiler_params=pltpu.CompilerParams(use_tc_tiling_on_sc=False, needs_layout_passes=False),
    )(idx_pad, alpha, m)


def _scatter_body(idx_hbm, w_hbm, e_hbm, z64_hbm, z2_hbm, num_hbm, s_hbm,
                  idx_v, w_v, e_v, accum, acc2, sem, sem2, sem3):
    cid = lax.axis_index("c")
    sid = lax.axis_index("s")
    iota16 = lax.iota(jnp.int32, 16)
    n_chunks = E_PAD // NS // GCH

    for q in range(2):
        base_node = cid * N_HALF + q * N_QTR

        zb = sid * (Q_ROWS // NS)
        pltpu.sync_copy(z64_hbm, accum.at[pl.ds(zb, GCH)])
        pltpu.sync_copy(z2_hbm, acc2.at[pl.ds(zb, GCH)])
        pltpu.sync_copy(z64_hbm.at[pl.ds(0, 304)], accum.at[pl.ds(zb + GCH, 304)])
        pltpu.sync_copy(z2_hbm.at[pl.ds(0, 304)], acc2.at[pl.ds(zb + GCH, 304)])
        plsc.subcore_barrier()

        def step(i, carry):
            base = (sid * (E_PAD // NS)) + i * GCH
            h1 = pltpu.async_copy(idx_hbm.at[pl.ds(base, GCH)], idx_v, sem)
            h2 = pltpu.async_copy(w_hbm.at[pl.ds(base, GCH)], w_v, sem2)
            h3 = pltpu.async_copy(e_hbm.at[pl.ds(base, GCH)], e_v, sem3)
            h1.wait()
            h2.wait()
            h3.wait()
            for v in range(GCH // 16):
                sl = pl.ds(v * 16, 16)
                d = idx_v[sl]
                local = d - base_node
                inb = (local >= 0) & (local < N_QTR)
                trash = (N_QTR + v * 16) + iota16
                idx_v[sl] = jnp.where(inb, local, trash)
            pltpu.sync_copy(w_v, accum.at[idx_v], add=True)
            pltpu.sync_copy(e_v, acc2.at[idx_v], add=True)
            return carry

        lax.fori_loop(0, n_chunks, step, 0)
        plsc.subcore_barrier()

        for k in range(2):
            ci = sid + k * NS

            @pl.when(ci < 25)
            def _():
                pltpu.sync_copy(accum.at[pl.ds(ci * 500, 500)],
                                num_hbm.at[pl.ds(base_node + ci * 500, 500)])
                pltpu.sync_copy(acc2.at[pl.ds(ci * 500, 500)],
                                s_hbm.at[pl.ds(base_node + ci * 500, 500)])
        if q == 0:
            plsc.subcore_barrier()


def _sc_scatter(idx_pad, w, e2):
    z64 = jnp.zeros((GCH, HC), jnp.float32)
    z2 = jnp.zeros((GCH, 2), jnp.float32)
    return pl.kernel(
        _scatter_body,
        out_type=(jax.ShapeDtypeStruct((N, HC), jnp.float32),
                  jax.ShapeDtypeStruct((N, 2), jnp.float32)),
        mesh=_sc_mesh(),
        scratch_types=[
            pltpu.VMEM((GCH,), jnp.int32),
            pltpu.VMEM((GCH, HC), jnp.float32),
            pltpu.VMEM((GCH, 2), jnp.float32),
            pltpu.VMEM_SHARED((Q_ROWS, HC), jnp.float32),
            pltpu.VMEM_SHARED((Q_ROWS, 2), jnp.float32),
            pltpu.SemaphoreType.DMA,
            pltpu.SemaphoreType.DMA,
            pltpu.SemaphoreType.DMA,
        ],
        compiler_params=pltpu.CompilerParams(use_tc_tiling_on_sc=False, needs_layout_passes=False),
    )(idx_pad, w, e2, z64, z2)


XL_R = 2000


def _xl_body(h_ref, w_ref, b_ref, out_ref):
    out_ref[...] = jnp.dot(h_ref[...], w_ref[...],
                           preferred_element_type=jnp.float32) + b_ref[...]


def _xl_pass(h, W, b):
    d_in = h.shape[1]
    return pl.pallas_call(
        _xl_body,
        grid=(N // XL_R,),
        in_specs=[
            pl.BlockSpec((XL_R, d_in), lambda i: (i, 0)),
            pl.BlockSpec((d_in, HC), lambda i: (0, 0)),
            pl.BlockSpec((1, HC), lambda i: (0, 0)),
        ],
        out_specs=pl.BlockSpec((XL_R, HC), lambda i: (i, 0)),
        out_shape=jax.ShapeDtypeStruct((N, HC), jnp.float32),
    )(h, W, b.reshape(1, HC))


AL_R = 8192


def _alpha_body(xs_ref, xd_ref, ea_ref, we_ref, att_ref, out_ref):
    i = pl.program_id(0)
    q = xs_ref[...] + xd_ref[...] + jnp.dot(ea_ref[...], we_ref[...],
                                            preferred_element_type=jnp.float32)
    g = jnp.where(q > 0, q, 0.2 * q)
    ga = g * att_ref[...]
    a0 = jnp.sum(ga[:, :C], axis=1, keepdims=True)
    a1 = jnp.sum(ga[:, C:], axis=1, keepdims=True)
    alpha = jnp.concatenate([a0, a1], axis=1)
    row = i * AL_R + lax.broadcasted_iota(jnp.int32, (AL_R, 2), 0)
    out_ref[...] = jnp.where(row < E_FULL, alpha, NEG)


def _alpha_pass(xs, xd, ea_pad, We, att):
    return pl.pallas_call(
        _alpha_body,
        grid=(E_PAD // AL_R,),
        in_specs=[
            pl.BlockSpec((AL_R, HC), lambda i: (i, 0)),
            pl.BlockSpec((AL_R, HC), lambda i: (i, 0)),
            pl.BlockSpec((AL_R, D_E), lambda i: (i, 0)),
            pl.BlockSpec((D_E, HC), lambda i: (0, 0)),
            pl.BlockSpec((1, HC), lambda i: (0, 0)),
        ],
        out_specs=pl.BlockSpec((AL_R, 2), lambda i: (i, 0)),
        out_shape=jax.ShapeDtypeStruct((E_PAD, 2), jnp.float32),
    )(xs, xd, ea_pad, We, att.reshape(1, HC))


def _w_body(xs_ref, e_ref, out_ref):
    xs = xs_ref[...]
    e = e_ref[...]
    e0 = jnp.broadcast_to(e[:, 0:1], (AL_R, C))
    e1 = jnp.broadcast_to(e[:, 1:2], (AL_R, C))
    out_ref[...] = xs * jnp.concatenate([e0, e1], axis=1)


def _w_pass(xs, e2):
    return pl.pallas_call(
        _w_body,
        grid=(E_PAD // AL_R,),
        in_specs=[
            pl.BlockSpec((AL_R, HC), lambda i: (i, 0)),
            pl.BlockSpec((AL_R, 2), lambda i: (i, 0)),
        ],
        out_specs=pl.BlockSpec((AL_R, HC), lambda i: (i, 0)),
        out_shape=jax.ShapeDtypeStruct((E_PAD, HC), jnp.float32),
    )(xs, e2)


def _combine_body(num_ref, s_ref, b_ref, out_ref):
    num = num_ref[...]
    s = s_ref[...]
    s0 = jnp.broadcast_to(s[:, 0:1], (XL_R, C))
    s1 = jnp.broadcast_to(s[:, 1:2], (XL_R, C))
    o = num / (jnp.concatenate([s0, s1], axis=1) + 1e-16) + b_ref[...]
    out_ref[...] = jnp.where(o > 0, o, jnp.exp(o) - 1.0)


def _combine_pass(num, s, bias):
    return pl.pallas_call(
        _combine_body,
        grid=(N // XL_R,),
        in_specs=[
            pl.BlockSpec((XL_R, HC), lambda i: (i, 0)),
            pl.BlockSpec((XL_R, 2), lambda i: (i, 0)),
            pl.BlockSpec((1, HC), lambda i: (0, 0)),
        ],
        out_specs=pl.BlockSpec((XL_R, HC), lambda i: (i, 0)),
        out_shape=jax.ShapeDtypeStruct((N, HC), jnp.float32),
    )(num, s, bias.reshape(1, HC))


def _mean_body(ea_ref, out_ref):
    @pl.when(pl.program_id(0) == 0)
    def _():
        out_ref[...] = jnp.zeros_like(out_ref)

    out_ref[...] += jnp.sum(ea_ref[...], axis=0, keepdims=True) * (1.0 / E)


def _mean_pass(ea):
    return pl.pallas_call(
        _mean_body,
        grid=(100,),
        in_specs=[pl.BlockSpec((E // 100, D_E), lambda i: (i, 0))],
        out_specs=pl.BlockSpec((1, D_E), lambda i: (0, 0)),
        out_shape=jax.ShapeDtypeStruct((1, D_E), jnp.float32),
    )(ea)


MLP_R = 8192


def _mlp_body(hs_ref, hd_ref, ea_ref, wa_ref, wb_ref, wd_ref, bias_ref, w2_ref, b2_ref, out_ref):
    acc = jnp.dot(hs_ref[...], wa_ref[...], preferred_element_type=jnp.float32)
    acc += jnp.dot(hd_ref[...], wb_ref[...], preferred_element_type=jnp.float32)
    acc += jnp.dot(ea_ref[...], wd_ref[...], preferred_element_type=jnp.float32)
    hid = jnp.maximum(acc + bias_ref[...], 0.0)
    res = jnp.dot(hid, w2_ref[...], preferred_element_type=jnp.float32)[:, 0] + b2_ref[0]
    out_ref[...] = res.reshape(1, 1, MLP_R)


def _edge_mlp(h_src, h_dst, ea, goal, Wm1, bm1, Wm2, bm2):
    wa = Wm1[0:HC]
    wb = Wm1[HC:2 * HC]
    wc = Wm1[2 * HC:3 * HC]
    wd = Wm1[3 * HC:]
    bias_eff = (bm1 + goal @ wc).reshape(1, 32)
    return pl.pallas_call(
        _mlp_body,
        grid=(E_MLP // MLP_R,),
        in_specs=[
            pl.BlockSpec((MLP_R, HC), lambda i: (i, 0)),
            pl.BlockSpec((MLP_R, HC), lambda i: (i, 0)),
            pl.BlockSpec((MLP_R, D_E), lambda i: (i, 0)),
            pl.BlockSpec((HC, 32), lambda i: (0, 0)),
            pl.BlockSpec((HC, 32), lambda i: (0, 0)),
            pl.BlockSpec((D_E, 32), lambda i: (0, 0)),
            pl.BlockSpec((1, 32), lambda i: (0, 0)),
            pl.BlockSpec((32, 1), lambda i: (0, 0)),
            pl.BlockSpec((1,), lambda i: (0,)),
        ],
        out_specs=pl.BlockSpec((1, 1, MLP_R), lambda i: (i, 0, 0)),
        out_shape=jax.ShapeDtypeStruct((E_MLP // MLP_R, 1, MLP_R), jnp.float32),
    )(h_src, h_dst, ea, wa, wb, wd, bias_eff, Wm2, bm2).reshape(E_MLP)


def _gatv2_layer(xl2d, src_pad, dst_pad, ea_pad, We, att, bias):
    xs = _sc_gather(xl2d, src_pad, E_PAD, GCH)
    xd = _sc_gather(xl2d, dst_pad, E_PAD, GCH)
    alpha = _alpha_pass(xs, xd, ea_pad, We, att)
    mparts = _sc_segmax(dst_pad, alpha)
    m = _maxred_pass(mparts)
    e2 = _sc_epass(dst_pad, alpha, m)
    w = _w_pass(xs, e2)
    num, s = _sc_scatter(dst_pad, w, e2)
    return _combine_pass(num, s, bias)


def kernel(x, edge_index, edge_attr, dest_index, W1_l, b1_l, We1, att1, bias1,
           W2_l, b2_l, We2, att2, bias2, Wm1, bm1, Wm2, bm2):
    src, dst = edge_index[0], edge_index[1]
    loop = jnp.arange(N, dtype=src.dtype)
    pad = (jnp.arange(E_PAD - E_FULL, dtype=src.dtype) * 61) % N
    src_pad = jnp.concatenate([src, loop, pad], axis=0)
    dst_pad = jnp.concatenate([dst, loop, pad], axis=0)
    ea_mean = _mean_pass(edge_attr)
    ea_pad = jnp.concatenate([
        edge_attr,
        jnp.broadcast_to(ea_mean, (N, D_E)),
        jnp.zeros((E_PAD - E_FULL, D_E), jnp.float32),
    ], axis=0)

    xl1 = _xl_pass(x, W1_l, b1_l)
    h = _gatv2_layer(xl1, src_pad, dst_pad, ea_pad, We1, att1.reshape(HC), bias1)
    xl2 = _xl_pass(h, W2_l, b2_l)
    h = _gatv2_layer(xl2, src_pad, dst_pad, ea_pad, We2, att2.reshape(HC), bias2)

    goal = h[dest_index]
    mpad = (jnp.arange(E_MLP - E, dtype=src.dtype) * 61) % N
    src_m = jnp.concatenate([src, mpad], axis=0)
    dst_m = jnp.concatenate([dst, mpad], axis=0)
    hs = _sc_gather(h, src_m, E_MLP, GCH)
    hd = _sc_gather(h, dst_m, E_MLP, GCH)
    ea_m = jnp.concatenate([edge_attr, jnp.zeros((E_MLP - E, D_E), jnp.float32)], axis=0)
    return _edge_mlp(hs, hd, ea_m, goal, Wm1, bm1, Wm2, bm2)[:E]
